# Initial kernel scaffold; baseline (speedup 1.0000x reference)
import functools

import jax
import jax.numpy as jnp
from jax import lax
from jax.experimental import pallas as pl
from jax.experimental.pallas import tpu as pltpu
from jax.experimental.pallas import tpu_sc as plsc

N = 10000
NP = 10240
E = 320000
CH = 128
NSUB = 16
CPT = 157
EP = NSUB * CPT * CH
W = 112
STRIPE = NP // NSUB
F32 = jnp.float32
I32 = jnp.int32


def _k1_body(x_ref, wp_ref, we_ref, t0_ref, t1_ref):
    xb = x_ref[...]
    tp = jnp.dot(xb, wp_ref[...], preferred_element_type=F32)
    te = jnp.dot(xb, we_ref[...], preferred_element_type=F32)
    nrows = xb.shape[0]
    t0_ref[...] = jnp.concatenate(
        [tp[:, :96], jnp.zeros((nrows, 16), F32)], axis=1)
    t1_ref[...] = jnp.concatenate(
        [tp[:, 96:], te, jnp.ones((nrows, 1), F32),
         jnp.zeros((nrows, 15), F32)], axis=1)


def _build_tables(xp, w1p_w, w1e_w):
    blk = 1280
    return pl.pallas_call(
        _k1_body,
        grid=(NP // blk,),
        in_specs=[
            pl.BlockSpec((blk, 5), lambda i: (i, 0)),
            pl.BlockSpec((5, 128), lambda i: (0, 0)),
            pl.BlockSpec((5, 64), lambda i: (0, 0)),
        ],
        out_specs=[
            pl.BlockSpec((blk, W), lambda i: (i, 0)),
            pl.BlockSpec((blk, W), lambda i: (i, 0)),
        ],
        out_shape=[
            jax.ShapeDtypeStruct((NP, W), F32),
            jax.ShapeDtypeStruct((NP, W), F32),
        ],
    )(xp, w1p_w, w1e_w)


def _sca_body(t0, t1, srcp, dstp, wp, agg0_o, agg1_o, dega_o,
              acc, dacc, rows, wrow, srcv, dstv, wv, sem):
    c = lax.axis_index("c")
    s = lax.axis_index("s")
    zf = jnp.zeros((16,), F32)
    zi = jnp.zeros((16,), I32)
    iota16 = lax.iota(I32, 16)

    def _zero_row(r, _):
        for j in range(W // 16):
            rows[r, pl.ds(j * 16, 16)] = zf
        wrow[r, :] = zf
        return 0
    lax.fori_loop(0, CH, _zero_row, 0)

    def _zero_acc(j, _):
        r0 = s * STRIPE + j * CH
        pltpu.sync_copy(rows, acc.at[pl.ds(r0, CH)])
        pltpu.sync_copy(wrow, dacc.at[pl.ds(r0, CH)])
        return 0
    lax.fori_loop(0, STRIPE // CH, _zero_acc, 0)
    plsc.subcore_barrier()

    def _chunk(g, _):
        base = (s * CPT + g) * CH
        pltpu.sync_copy(srcp.at[pl.ds(base, CH)], srcv)
        pltpu.sync_copy(dstp.at[pl.ds(base, CH)], dstv)

        @pl.when(c == 0)
        def _deg():
            pltpu.sync_copy(wp.at[pl.ds(base, CH)], wv)
            for j in range(CH // 16):
                w16 = wv[pl.ds(j * 16, 16)]
                plsc.store_scatter(wrow, [iota16 + (j * 16), zi], w16)
            pltpu.sync_copy(wrow, dacc.at[srcv], add=True)

        @pl.when(c == 0)
        def _g0():
            pltpu.async_copy(t0.at[srcv], rows, sem).wait()

        @pl.when(c == 1)
        def _g1():
            pltpu.async_copy(t1.at[srcv], rows, sem).wait()

        pltpu.sync_copy(rows, acc.at[dstv], add=True)
        return 0
    lax.fori_loop(0, CPT, _chunk, 0)
    plsc.subcore_barrier()

    def _flush(j, _):
        r0 = s * STRIPE + j * CH

        @pl.when(c == 0)
        def _f0():
            pltpu.sync_copy(acc.at[pl.ds(r0, CH)], agg0_o.at[pl.ds(r0, CH)])
            pltpu.sync_copy(dacc.at[pl.ds(r0, CH)], dega_o.at[pl.ds(r0, CH)])

        @pl.when(c == 1)
        def _f1():
            pltpu.sync_copy(acc.at[pl.ds(r0, CH)], agg1_o.at[pl.ds(r0, CH)])
        return 0
    lax.fori_loop(0, STRIPE // CH, _flush, 0)


def _run_sca(t0, t1, srcp, dstp, wp):
    mesh = plsc.VectorSubcoreMesh(core_axis_name="c", subcore_axis_name="s")
    f = pl.kernel(
        _sca_body, mesh=mesh,
        out_type=[
            jax.ShapeDtypeStruct((NP, W), F32),
            jax.ShapeDtypeStruct((NP, W), F32),
            jax.ShapeDtypeStruct((NP, 16), F32),
        ],
        scratch_types=[
            pltpu.VMEM_SHARED((NP, W), F32),
            pltpu.VMEM_SHARED((NP, 16), F32),
            pltpu.VMEM((CH, W), F32),
            pltpu.VMEM((CH, 16), F32),
            pltpu.VMEM((CH,), I32),
            pltpu.VMEM((CH,), I32),
            pltpu.VMEM((CH,), F32),
            pltpu.SemaphoreType.DMA,
        ],
    )
    return f(t0, t1, srcp, dstp, wp)


def _k2_body(a0_ref, a1_ref, dega_ref, x_ref, wpr_ref, bp_ref, wer_ref,
             be_ref, ds0_ref, ds1_ref, out_ref, ss_ref):
    i = pl.program_id(0)
    a0 = a0_ref[...]
    a1 = a1_ref[...]
    x = x_ref[...]
    cnt = a1[:, 96:97]
    deg = dega_ref[...][:, 0:1]
    cntc = jnp.maximum(cnt, 1.0)
    aggP = jnp.concatenate([a0[:, :96], a1[:, :32]], axis=1) / cntc
    mat_s = aggP + jnp.dot(x, wpr_ref[...], preferred_element_type=F32) \
        + bp_ref[...]
    m = jnp.max(mat_s, axis=1, keepdims=True)
    e = jnp.exp(mat_s - m)
    S = e / jnp.sum(e, axis=1, keepdims=True)
    mat_y = jnp.maximum(
        a1[:, 32:96] / cntc
        + jnp.dot(x, wer_ref[...], preferred_element_type=F32)
        + be_ref[...], 0.0)
    dinv = jnp.where(deg > 0.0, 1.0 / jnp.sqrt(jnp.maximum(deg, 1e-12)), 0.0)
    DS = dinv * S
    ds0_ref[...] = DS[:, :64]
    ds1_ref[...] = DS[:, 64:]

    @pl.when(i == 0)
    def _init():
        out_ref[...] = jnp.zeros_like(out_ref)
        ss_ref[0, 0] = 0.0

    out_ref[...] += lax.dot_general(
        S, mat_y, (((0,), (0,)), ((), ())), preferred_element_type=F32)
    ss_ref[0, 0] += jnp.sum(S * S)


def _run_k2(agg0, agg1, dega, xp, w1p_root, b1p, w1e_root, b1e):
    blk = 1000
    return pl.pallas_call(
        _k2_body,
        grid=(N // blk,),
        in_specs=[
            pl.BlockSpec((blk, W), lambda i: (i, 0)),
            pl.BlockSpec((blk, W), lambda i: (i, 0)),
            pl.BlockSpec((blk, 16), lambda i: (i, 0)),
            pl.BlockSpec((blk, 5), lambda i: (i, 0)),
            pl.BlockSpec((5, 128), lambda i: (0, 0)),
            pl.BlockSpec((1, 128), lambda i: (0, 0)),
            pl.BlockSpec((5, 64), lambda i: (0, 0)),
            pl.BlockSpec((1, 64), lambda i: (0, 0)),
        ],
        out_specs=[
            pl.BlockSpec((blk, 64), lambda i: (i, 0)),
            pl.BlockSpec((blk, 64), lambda i: (i, 0)),
            pl.BlockSpec((128, 64), lambda i: (0, 0)),
            pl.BlockSpec(memory_space=pltpu.SMEM),
        ],
        out_shape=[
            jax.ShapeDtypeStruct((NP, 64), F32),
            jax.ShapeDtypeStruct((NP, 64), F32),
            jax.ShapeDtypeStruct((128, 64), F32),
            jax.ShapeDtypeStruct((1, 1), F32),
        ],
    )(agg0, agg1, dega, xp, w1p_root, b1p, w1e_root, b1e)


def _scb_body(ds0, ds1, srcp, dstp, wp, z0_o, z1_o,
              zacc, rows, srcv, dstv, wv, sem):
    c = lax.axis_index("c")
    s = lax.axis_index("s")
    zf = jnp.zeros((16,), F32)
    zi = jnp.zeros((16,), I32)
    iota16 = lax.iota(I32, 16)

    def _zero_row(r, _):
        for j in range(4):
            rows[r, pl.ds(j * 16, 16)] = zf
        return 0
    lax.fori_loop(0, CH, _zero_row, 0)

    def _zero_acc(j, _):
        pltpu.sync_copy(rows, zacc.at[pl.ds(s * STRIPE + j * CH, CH)])
        return 0
    lax.fori_loop(0, STRIPE // CH, _zero_acc, 0)
    plsc.subcore_barrier()

    def _chunk(g, _):
        base = (s * CPT + g) * CH
        pltpu.sync_copy(srcp.at[pl.ds(base, CH)], srcv)
        pltpu.sync_copy(dstp.at[pl.ds(base, CH)], dstv)
        pltpu.sync_copy(wp.at[pl.ds(base, CH)], wv)

        @pl.when(c == 0)
        def _g0():
            pltpu.async_copy(ds0.at[dstv], rows, sem).wait()

        @pl.when(c == 1)
        def _g1():
            pltpu.async_copy(ds1.at[dstv], rows, sem).wait()

        def _scale(j, _):
            w16 = wv[pl.ds(j * 16, 16)]
            rid = iota16 + j * 16
            for k in range(64):
                cidx = zi + k
                v = plsc.load_gather(rows, [rid, cidx])
                plsc.store_scatter(rows, [rid, cidx], v * w16)
            return 0
        lax.fori_loop(0, CH // 16, _scale, 0)

        pltpu.sync_copy(rows, zacc.at[srcv], add=True)
        return 0
    lax.fori_loop(0, CPT, _chunk, 0)
    plsc.subcore_barrier()

    def _flush(j, _):
        r0 = s * STRIPE + j * CH

        @pl.when(c == 0)
        def _f0():
            pltpu.sync_copy(zacc.at[pl.ds(r0, CH)], z0_o.at[pl.ds(r0, CH)])

        @pl.when(c == 1)
        def _f1():
            pltpu.sync_copy(zacc.at[pl.ds(r0, CH)], z1_o.at[pl.ds(r0, CH)])
        return 0
    lax.fori_loop(0, STRIPE // CH, _flush, 0)


def _run_scb(ds0, ds1, srcp, dstp, wp):
    mesh = plsc.VectorSubcoreMesh(core_axis_name="c", subcore_axis_name="s")
    f = pl.kernel(
        _scb_body, mesh=mesh,
        out_type=[
            jax.ShapeDtypeStruct((NP, 64), F32),
            jax.ShapeDtypeStruct((NP, 64), F32),
        ],
        scratch_types=[
            pltpu.VMEM_SHARED((NP, 64), F32),
            pltpu.VMEM((CH, 64), F32),
            pltpu.VMEM((CH,), I32),
            pltpu.VMEM((CH,), I32),
            pltpu.VMEM((CH,), F32),
            pltpu.SemaphoreType.DMA,
        ],
    )
    return f(ds0, ds1, srcp, dstp, wp)


def _k3_body(ds0_ref, ds1_ref, z0_ref, z1_ref, out_ref, ss_ref,
             w2ew_ref, w2er_ref, b2e_ref, l1w_ref, l1b_ref, l2w_ref,
             l2b_ref, logp_ref, reg_ref):
    i = pl.program_id(0)
    ng = pl.num_programs(0)

    @pl.when(i == 0)
    def _init():
        reg_ref[0, 0] = 0.0

    reg_ref[0, 0] += (jnp.sum(ds0_ref[...] * z0_ref[...])
                      + jnp.sum(ds1_ref[...] * z1_ref[...]))

    @pl.when(i == ng - 1)
    def _final():
        reg_ref[0, 0] = (ss_ref[0, 0] - reg_ref[0, 0]) / float(N)
        o = out_ref[...]
        t = jnp.dot(o, w2ew_ref[...], preferred_element_type=F32)
        me = jnp.sum(t, axis=0, keepdims=True) / 128.0
        my2 = jnp.maximum(
            me + jnp.dot(o, w2er_ref[...], preferred_element_type=F32)
            + b2e_ref[...], 0.0)
        out2 = jnp.sum(my2, axis=0, keepdims=True)
        h = jnp.maximum(
            jnp.dot(out2, l1w_ref[...], preferred_element_type=F32)
            + l1b_ref[...], 0.0)
        lg = jnp.dot(h, l2w_ref[...], preferred_element_type=F32) \
            + l2b_ref[...]
        m = jnp.max(lg)
        logp_ref[...] = lg - (m + jnp.log(jnp.sum(jnp.exp(lg - m))))


def _run_k3(ds0, ds1, z0, z1, out128, ss, w2e_w, w2e_root, b2e,
            lin1_w, lin1_b, lin2_w, lin2_b):
    blk = 1000
    return pl.pallas_call(
        _k3_body,
        grid=(N // blk,),
        in_specs=[
            pl.BlockSpec((blk, 64), lambda i: (i, 0)),
            pl.BlockSpec((blk, 64), lambda i: (i, 0)),
            pl.BlockSpec((blk, 64), lambda i: (i, 0)),
            pl.BlockSpec((blk, 64), lambda i: (i, 0)),
            pl.BlockSpec((128, 64), lambda i: (0, 0)),
            pl.BlockSpec(memory_space=pltpu.SMEM),
            pl.BlockSpec((64, 64), lambda i: (0, 0)),
            pl.BlockSpec((64, 64), lambda i: (0, 0)),
            pl.BlockSpec((1, 64), lambda i: (0, 0)),
            pl.BlockSpec((64, 256), lambda i: (0, 0)),
            pl.BlockSpec((1, 256), lambda i: (0, 0)),
            pl.BlockSpec((256, 8), lambda i: (0, 0)),
            pl.BlockSpec((1, 8), lambda i: (0, 0)),
        ],
        out_specs=[
            pl.BlockSpec((1, 8), lambda i: (0, 0)),
            pl.BlockSpec(memory_space=pltpu.SMEM),
        ],
        out_shape=[
            jax.ShapeDtypeStruct((1, 8), F32),
            jax.ShapeDtypeStruct((1, 1), F32),
        ],
    )(ds0, ds1, z0, z1, out128, ss, w2e_w, w2e_root, b2e,
      lin1_w, lin1_b, lin2_w, lin2_b)


def kernel(x, edge_index, edge_wht,
           w1p_w, w1p_root, w1p_b, w1e_w, w1e_root, w1e_b,
           w2p_w, w2p_root, w2p_b, w2e_w, w2e_root, w2e_b,
           lin1_w, lin1_b, lin2_w, lin2_b):
    src = edge_index[0]
    dst = edge_index[1]
    w = edge_wht.reshape(-1)
    pad = EP - E
    srcp = jnp.concatenate([src, jnp.full((pad,), N, src.dtype)]).astype(I32)
    dstp = jnp.concatenate([dst, jnp.full((pad,), N, dst.dtype)]).astype(I32)
    wp = jnp.concatenate([w, jnp.zeros((pad,), F32)])
    xp = jnp.pad(x, ((0, NP - N), (0, 0)))

    t0, t1 = _build_tables(xp, w1p_w, w1e_w)
    agg0, agg1, dega = _run_sca(t0, t1, srcp, dstp, wp)
    ds0, ds1, out128, ss = _run_k2(
        agg0, agg1, dega, xp, w1p_root, w1p_b.reshape(1, 128),
        w1e_root, w1e_b.reshape(1, 64))
    z0, z1 = _run_scb(ds0, ds1, srcp, dstp, wp)
    logp, reg = _run_k3(
        ds0, ds1, z0, z1, out128, ss, w2e_w, w2e_root,
        w2e_b.reshape(1, 64), lin1_w, lin1_b.reshape(1, 256),
        lin2_w, lin2_b.reshape(1, 8))
    return (logp, reg.reshape(()))


# speedup vs baseline: 2.4189x; 2.4189x over previous
import jax
import jax.numpy as jnp
from jax import lax
from jax.experimental import pallas as pl
from jax.experimental.pallas import tpu as pltpu
from jax.experimental.pallas import tpu_sc as plsc

N = 10000
NP = 10240
E = 320000
CH = 128
NSUB = 16
CPT = 158
CPW = 79
EP = NSUB * CPT * CH
W = 128
STRIPE = NP // NSUB
F32 = jnp.float32
I32 = jnp.int32


def _k1_body(x_ref, wp_ref, we_ref, tt_ref):
    i = pl.program_id(0)
    xb = x_ref[...]
    tp = jnp.dot(xb, wp_ref[...], preferred_element_type=F32)
    te = jnp.dot(xb, we_ref[...], preferred_element_type=F32)
    nrows = xb.shape[0]
    t1 = jnp.concatenate(
        [te, jnp.ones((nrows, 1), F32), jnp.zeros((nrows, 63), F32)], axis=1)
    tt_ref[...] = jnp.where(i < NP // 1280, tp, t1)


def _build_tables(xp, w1p_w, w1e_w):
    blk = 1280
    nb = NP // blk
    return pl.pallas_call(
        _k1_body,
        grid=(2 * nb,),
        in_specs=[
            pl.BlockSpec((blk, 5), lambda i: (lax.rem(i, nb), 0)),
            pl.BlockSpec((5, 128), lambda i: (0, 0)),
            pl.BlockSpec((5, 64), lambda i: (0, 0)),
        ],
        out_specs=[
            pl.BlockSpec((blk, W), lambda i: (i, 0)),
        ],
        out_shape=[
            jax.ShapeDtypeStruct((2 * NP, W), F32),
        ],
    )(xp, w1p_w, w1e_w)


DCOL = 80


def _sca_body(tt, srcp, srcp2, dstp, wmat, agg0_o, agg1_o,
              acc, rows, wrow, wfull, srcv, srcv2, dstv, sem):
    c = lax.axis_index("c")
    s = lax.axis_index("s")
    zf = jnp.zeros((16,), F32)

    def _zero_row(r, _):
        for j in range(W // 16):
            rows[r, pl.ds(j * 16, 16)] = zf
        return 0
    lax.fori_loop(0, CH, _zero_row, 0)

    def _zero_wf(r, _):
        for j in range(W // 16):
            wfull[r, pl.ds(j * 16, 16)] = zf
        return 0
    lax.fori_loop(0, 32, _zero_wf, 0)

    def _zero_acc(j, _):
        r0 = s * STRIPE + j * CH
        pltpu.sync_copy(rows, acc.at[pl.ds(r0, CH)])
        return 0
    lax.fori_loop(0, STRIPE // CH, _zero_acc, 0)
    plsc.subcore_barrier()

    coff = c * NP

    def _chunk(g, _):
        base = (s * CPT + g) * CH
        pltpu.sync_copy(srcp.at[pl.ds(base, CH)], srcv)
        pltpu.sync_copy(dstp.at[pl.ds(base, CH)], dstv)

        @pl.when(c == 1)
        def _deg():
            pltpu.sync_copy(srcp2.at[pl.ds((s * CPT + g) * 4, 4)], srcv2)
            for q in range(4):
                pltpu.sync_copy(wmat.at[pl.ds(base + q * 32, 32)], wrow)

                def _exp(r, _):
                    wfull[r, pl.ds(DCOL, 16)] = wrow[r, :]
                    return 0
                lax.fori_loop(0, 32, _exp, 0)
                pltpu.sync_copy(wfull, acc.at[srcv2.at[q]], add=True)

        for j in range(CH // 16):
            v = srcv[pl.ds(j * 16, 16)]
            srcv[pl.ds(j * 16, 16)] = v + coff
        pltpu.async_copy(tt.at[srcv], rows, sem).wait()

        pltpu.sync_copy(rows, acc.at[dstv], add=True)
        return 0
    lax.fori_loop(0, CPT, _chunk, 0)
    plsc.subcore_barrier()

    def _flush(j, _):
        r0 = s * STRIPE + j * CH

        @pl.when(c == 0)
        def _f0():
            pltpu.sync_copy(acc.at[pl.ds(r0, CH)], agg0_o.at[pl.ds(r0, CH)])

        @pl.when(c == 1)
        def _f1():
            pltpu.sync_copy(acc.at[pl.ds(r0, CH)], agg1_o.at[pl.ds(r0, CH)])
        return 0
    lax.fori_loop(0, STRIPE // CH, _flush, 0)


def _run_sca(tt, srcp, dstp, wmat):
    mesh = plsc.VectorSubcoreMesh(core_axis_name="c", subcore_axis_name="s")
    f = pl.kernel(
        _sca_body, mesh=mesh,
        out_type=[
            jax.ShapeDtypeStruct((NP, W), F32),
            jax.ShapeDtypeStruct((NP, W), F32),
        ],
        scratch_types=[
            pltpu.VMEM_SHARED((NP, W), F32),
            pltpu.VMEM((CH, W), F32),
            pltpu.VMEM((32, 16), F32),
            pltpu.VMEM((32, W), F32),
            pltpu.VMEM((CH,), I32),
            pltpu.VMEM((4, 32), I32),
            pltpu.VMEM((CH,), I32),
            pltpu.SemaphoreType.DMA,
        ],
    )
    return f(tt, srcp, srcp.reshape(EP // 32, 32), dstp, wmat)


def _k2_body(a0_ref, a1_ref, x_ref, wpr_ref, bp_ref, wer_ref,
             be_ref, ds_ref, out_ref, ss_ref):
    i = pl.program_id(0)
    a0 = a0_ref[...]
    a1 = a1_ref[...]
    x = x_ref[...]
    cnt = a1[:, 64:65]
    deg = a1[:, DCOL:DCOL + 1]
    cntc = jnp.maximum(cnt, 1.0)
    mat_s = a0 / cntc + jnp.dot(x, wpr_ref[...], preferred_element_type=F32) \
        + bp_ref[...]
    m = jnp.max(mat_s, axis=1, keepdims=True)
    e = jnp.exp(mat_s - m)
    S = e / jnp.sum(e, axis=1, keepdims=True)
    mat_y = jnp.maximum(
        a1[:, :64] / cntc
        + jnp.dot(x, wer_ref[...], preferred_element_type=F32)
        + be_ref[...], 0.0)
    dinv = jnp.where(deg > 0.0, 1.0 / jnp.sqrt(jnp.maximum(deg, 1e-12)), 0.0)
    ds_ref[...] = dinv * S

    @pl.when(i == 0)
    def _init():
        out_ref[...] = jnp.zeros_like(out_ref)
        ss_ref[0, 0] = 0.0

    out_ref[...] += lax.dot_general(
        S, mat_y, (((0,), (0,)), ((), ())), preferred_element_type=F32)
    ss_ref[0, 0] += jnp.sum(S * S)


def _run_k2(agg0, agg1, xp, w1p_root, b1p, w1e_root, b1e):
    blk = 1000
    return pl.pallas_call(
        _k2_body,
        grid=(N // blk,),
        in_specs=[
            pl.BlockSpec((blk, W), lambda i: (i, 0)),
            pl.BlockSpec((blk, W), lambda i: (i, 0)),
            pl.BlockSpec((blk, 5), lambda i: (i, 0)),
            pl.BlockSpec((5, 128), lambda i: (0, 0)),
            pl.BlockSpec((1, 128), lambda i: (0, 0)),
            pl.BlockSpec((5, 64), lambda i: (0, 0)),
            pl.BlockSpec((1, 64), lambda i: (0, 0)),
        ],
        out_specs=[
            pl.BlockSpec((blk, 128), lambda i: (i, 0)),
            pl.BlockSpec((128, 64), lambda i: (0, 0)),
            pl.BlockSpec(memory_space=pltpu.SMEM),
        ],
        out_shape=[
            jax.ShapeDtypeStruct((NP, 128), F32),
            jax.ShapeDtypeStruct((128, 64), F32),
            jax.ShapeDtypeStruct((1, 1), F32),
        ],
    )(agg0, agg1, xp, w1p_root, b1p, w1e_root, b1e)


def _scb_body(ds, srcp, dstp, wmat, za_o, zb_o,
              zacc, rows, srcv, dstv, wvm, sem):
    c = lax.axis_index("c")
    s = lax.axis_index("s")
    zf = jnp.zeros((16,), F32)

    def _zero_row(r, _):
        for j in range(8):
            rows[r, pl.ds(j * 16, 16)] = zf
        return 0
    lax.fori_loop(0, CH, _zero_row, 0)

    def _zero_acc(j, _):
        pltpu.sync_copy(rows, zacc.at[pl.ds(s * STRIPE + j * CH, CH)])
        return 0
    lax.fori_loop(0, STRIPE // CH, _zero_acc, 0)
    plsc.subcore_barrier()

    wid = s * 2 + c

    def _chunk(g, _):
        base = (wid * CPW + g) * CH
        pltpu.sync_copy(srcp.at[pl.ds(base, CH)], srcv)
        pltpu.sync_copy(dstp.at[pl.ds(base, CH)], dstv)
        pltpu.sync_copy(wmat.at[pl.ds(base, CH)], wvm)
        pltpu.async_copy(ds.at[dstv], rows, sem).wait()

        def _scale(t, _):
            for u in range(4):
                e = t * 4 + u
                spl = wvm[e, :]
                for k in range(8):
                    v = rows[e, pl.ds(k * 16, 16)]
                    rows[e, pl.ds(k * 16, 16)] = v * spl
            return 0
        lax.fori_loop(0, CH // 4, _scale, 0)

        pltpu.sync_copy(rows, zacc.at[srcv], add=True)
        return 0
    lax.fori_loop(0, CPW, _chunk, 0)
    plsc.subcore_barrier()

    def _flush(j, _):
        r0 = s * STRIPE + j * CH

        @pl.when(c == 0)
        def _f0():
            pltpu.sync_copy(zacc.at[pl.ds(r0, CH)], za_o.at[pl.ds(r0, CH)])

        @pl.when(c == 1)
        def _f1():
            pltpu.sync_copy(zacc.at[pl.ds(r0, CH)], zb_o.at[pl.ds(r0, CH)])
        return 0
    lax.fori_loop(0, STRIPE // CH, _flush, 0)


def _run_scb(ds, srcp, dstp, wmat):
    mesh = plsc.VectorSubcoreMesh(core_axis_name="c", subcore_axis_name="s")
    f = pl.kernel(
        _scb_body, mesh=mesh,
        out_type=[
            jax.ShapeDtypeStruct((NP, 128), F32),
            jax.ShapeDtypeStruct((NP, 128), F32),
        ],
        scratch_types=[
            pltpu.VMEM_SHARED((NP, 128), F32),
            pltpu.VMEM((CH, 128), F32),
            pltpu.VMEM((CH,), I32),
            pltpu.VMEM((CH,), I32),
            pltpu.VMEM((CH, 16), F32),
            pltpu.SemaphoreType.DMA,
        ],
    )
    return f(ds, srcp, dstp, wmat)


def _k3_body(ds_ref, za_ref, zb_ref, out_ref, ss_ref,
             w2ew_ref, w2er_ref, b2e_ref, l1w_ref, l1b_ref, l2w_ref,
             l2b_ref, logp_ref, reg_ref):
    i = pl.program_id(0)
    ng = pl.num_programs(0)

    @pl.when(i == 0)
    def _init():
        reg_ref[0, 0] = 0.0

    reg_ref[0, 0] += jnp.sum(ds_ref[...] * (za_ref[...] + zb_ref[...]))

    @pl.when(i == ng - 1)
    def _final():
        reg_ref[0, 0] = (ss_ref[0, 0] - reg_ref[0, 0]) / float(N)
        o = out_ref[...]
        t = jnp.dot(o, w2ew_ref[...], preferred_element_type=F32)
        me = jnp.sum(t, axis=0, keepdims=True) / 128.0
        my2 = jnp.maximum(
            me + jnp.dot(o, w2er_ref[...], preferred_element_type=F32)
            + b2e_ref[...], 0.0)
        out2 = jnp.sum(my2, axis=0, keepdims=True)
        h = jnp.maximum(
            jnp.dot(out2, l1w_ref[...], preferred_element_type=F32)
            + l1b_ref[...], 0.0)
        lg = jnp.dot(h, l2w_ref[...], preferred_element_type=F32) \
            + l2b_ref[...]
        m = jnp.max(lg)
        logp_ref[...] = lg - (m + jnp.log(jnp.sum(jnp.exp(lg - m))))


def _run_k3(ds, za, zb, out128, ss, w2e_w, w2e_root, b2e,
            lin1_w, lin1_b, lin2_w, lin2_b):
    blk = 1000
    return pl.pallas_call(
        _k3_body,
        grid=(N // blk,),
        in_specs=[
            pl.BlockSpec((blk, 128), lambda i: (i, 0)),
            pl.BlockSpec((blk, 128), lambda i: (i, 0)),
            pl.BlockSpec((blk, 128), lambda i: (i, 0)),
            pl.BlockSpec((128, 64), lambda i: (0, 0)),
            pl.BlockSpec(memory_space=pltpu.SMEM),
            pl.BlockSpec((64, 64), lambda i: (0, 0)),
            pl.BlockSpec((64, 64), lambda i: (0, 0)),
            pl.BlockSpec((1, 64), lambda i: (0, 0)),
            pl.BlockSpec((64, 256), lambda i: (0, 0)),
            pl.BlockSpec((1, 256), lambda i: (0, 0)),
            pl.BlockSpec((256, 8), lambda i: (0, 0)),
            pl.BlockSpec((1, 8), lambda i: (0, 0)),
        ],
        out_specs=[
            pl.BlockSpec((1, 8), lambda i: (0, 0)),
            pl.BlockSpec(memory_space=pltpu.SMEM),
        ],
        out_shape=[
            jax.ShapeDtypeStruct((1, 8), F32),
            jax.ShapeDtypeStruct((1, 1), F32),
        ],
    )(ds, za, zb, out128, ss, w2e_w, w2e_root, b2e,
      lin1_w, lin1_b, lin2_w, lin2_b)


def kernel(x, edge_index, edge_wht,
           w1p_w, w1p_root, w1p_b, w1e_w, w1e_root, w1e_b,
           w2p_w, w2p_root, w2p_b, w2e_w, w2e_root, w2e_b,
           lin1_w, lin1_b, lin2_w, lin2_b):
    src = edge_index[0]
    dst = edge_index[1]
    w = edge_wht.reshape(-1)
    pad = EP - E
    srcp = jnp.concatenate([src, jnp.full((pad,), N, src.dtype)]).astype(I32)
    dstp = jnp.concatenate([dst, jnp.full((pad,), N, dst.dtype)]).astype(I32)
    wp = jnp.concatenate([w, jnp.zeros((pad,), F32)])
    wmat = jnp.broadcast_to(wp[:, None], (EP, 16)) * jnp.ones((EP, 16), F32)
    xp = jnp.pad(x, ((0, NP - N), (0, 0)))

    (tt,) = _build_tables(xp, w1p_w, w1e_w)
    agg0, agg1 = _run_sca(tt, srcp, dstp, wmat)
    ds, out128, ss = _run_k2(
        agg0, agg1, xp, w1p_root, w1p_b.reshape(1, 128),
        w1e_root, w1e_b.reshape(1, 64))
    za, zb = _run_scb(ds, srcp, dstp, wmat)
    logp, reg = _run_k3(
        ds, za, zb, out128, ss, w2e_w, w2e_root,
        w2e_b.reshape(1, 64), lin1_w, lin1_b.reshape(1, 256),
        lin2_w, lin2_b.reshape(1, 8))
    return (logp, reg.reshape(()))


# speedup vs baseline: 2.7606x; 1.1413x over previous
import jax
import jax.numpy as jnp
from jax import lax
from jax.experimental import pallas as pl
from jax.experimental.pallas import tpu as pltpu
from jax.experimental.pallas import tpu_sc as plsc

N = 10000
NP = 10240
E = 320000
CH = 128
NSUB = 16
CPT = 158
CHB = 64
CPWB = 158
EP = NSUB * CPT * CH
W = 128
STRIPE = NP // NSUB
F32 = jnp.float32
I32 = jnp.int32


def _k1_body(x_ref, wp_ref, we_ref, tt_ref):
    i = pl.program_id(0)
    xb = x_ref[...]
    tp = jnp.dot(xb, wp_ref[...], preferred_element_type=F32)
    te = jnp.dot(xb, we_ref[...], preferred_element_type=F32)
    nrows = xb.shape[0]
    t1 = jnp.concatenate(
        [te, jnp.ones((nrows, 1), F32), jnp.zeros((nrows, 63), F32)], axis=1)
    tt_ref[...] = jnp.where(i < NP // 1280, tp, t1)


def _build_tables(xp, w1p_w, w1e_w):
    blk = 1280
    nb = NP // blk
    return pl.pallas_call(
        _k1_body,
        grid=(2 * nb,),
        in_specs=[
            pl.BlockSpec((blk, 5), lambda i: (lax.rem(i, nb), 0)),
            pl.BlockSpec((5, 128), lambda i: (0, 0)),
            pl.BlockSpec((5, 64), lambda i: (0, 0)),
        ],
        out_specs=[
            pl.BlockSpec((blk, W), lambda i: (i, 0)),
        ],
        out_shape=[
            jax.ShapeDtypeStruct((2 * NP, W), F32),
        ],
    )(xp, w1p_w, w1e_w)


DCOL = 80


def _sca_body(tt, srcp, srcp2, dstp, wmat, agg0_o, agg1_o,
              acc, rows, wrow, wfull, srcv, srcv2, dstv, sem):
    c = lax.axis_index("c")
    s = lax.axis_index("s")
    zf = jnp.zeros((16,), F32)

    def _zero_row(r, _):
        for j in range(W // 16):
            rows[r, pl.ds(j * 16, 16)] = zf
        return 0
    lax.fori_loop(0, CH, _zero_row, 0)

    def _zero_wf(r, _):
        for j in range(W // 16):
            wfull[r, pl.ds(j * 16, 16)] = zf
        return 0
    lax.fori_loop(0, 32, _zero_wf, 0)

    def _zero_acc(j, _):
        r0 = s * STRIPE + j * CH
        pltpu.sync_copy(rows, acc.at[pl.ds(r0, CH)])
        return 0
    lax.fori_loop(0, STRIPE // CH, _zero_acc, 0)
    plsc.subcore_barrier()

    coff = c * NP

    def _chunk(g, _):
        base = (s * CPT + g) * CH
        pltpu.sync_copy(srcp.at[pl.ds(base, CH)], srcv)
        pltpu.sync_copy(dstp.at[pl.ds(base, CH)], dstv)

        @pl.when(c == 1)
        def _deg():
            pltpu.sync_copy(srcp2.at[pl.ds((s * CPT + g) * 4, 4)], srcv2)
            for q in range(4):
                pltpu.sync_copy(wmat.at[pl.ds(base + q * 32, 32)], wrow)

                def _exp(r, _):
                    wfull[r, pl.ds(DCOL, 16)] = wrow[r, :]
                    return 0
                lax.fori_loop(0, 32, _exp, 0)
                pltpu.sync_copy(wfull, acc.at[srcv2.at[q]], add=True)

        for j in range(CH // 16):
            v = srcv[pl.ds(j * 16, 16)]
            srcv[pl.ds(j * 16, 16)] = v + coff
        pltpu.async_copy(tt.at[srcv], rows, sem).wait()

        pltpu.sync_copy(rows, acc.at[dstv], add=True)
        return 0
    lax.fori_loop(0, CPT, _chunk, 0)
    plsc.subcore_barrier()

    def _flush(j, _):
        r0 = s * STRIPE + j * CH

        @pl.when(c == 0)
        def _f0():
            pltpu.sync_copy(acc.at[pl.ds(r0, CH)], agg0_o.at[pl.ds(r0, CH)])

        @pl.when(c == 1)
        def _f1():
            pltpu.sync_copy(acc.at[pl.ds(r0, CH)], agg1_o.at[pl.ds(r0, CH)])
        return 0
    lax.fori_loop(0, STRIPE // CH, _flush, 0)


def _run_sca(tt, srcp, dstp, wmat):
    mesh = plsc.VectorSubcoreMesh(core_axis_name="c", subcore_axis_name="s")
    f = pl.kernel(
        _sca_body, mesh=mesh,
        out_type=[
            jax.ShapeDtypeStruct((NP, W), F32),
            jax.ShapeDtypeStruct((NP, W), F32),
        ],
        scratch_types=[
            pltpu.VMEM_SHARED((NP, W), F32),
            pltpu.VMEM((CH, W), F32),
            pltpu.VMEM((32, 16), F32),
            pltpu.VMEM((32, W), F32),
            pltpu.VMEM((CH,), I32),
            pltpu.VMEM((4, 32), I32),
            pltpu.VMEM((CH,), I32),
            pltpu.SemaphoreType.DMA,
        ],
    )
    return f(tt, srcp, srcp.reshape(EP // 32, 32), dstp, wmat)


def _k2_body(a0_ref, a1_ref, x_ref, wpr_ref, bp_ref, wer_ref,
             be_ref, ds_ref, out_ref, ss_ref):
    i = pl.program_id(0)
    a0 = a0_ref[...]
    a1 = a1_ref[...]
    x = x_ref[...]
    cnt = a1[:, 64:65]
    deg = a1[:, DCOL:DCOL + 1]
    cntc = jnp.maximum(cnt, 1.0)
    mat_s = a0 / cntc + jnp.dot(x, wpr_ref[...], preferred_element_type=F32) \
        + bp_ref[...]
    m = jnp.max(mat_s, axis=1, keepdims=True)
    e = jnp.exp(mat_s - m)
    S = e / jnp.sum(e, axis=1, keepdims=True)
    mat_y = jnp.maximum(
        a1[:, :64] / cntc
        + jnp.dot(x, wer_ref[...], preferred_element_type=F32)
        + be_ref[...], 0.0)
    dinv = jnp.where(deg > 0.0, 1.0 / jnp.sqrt(jnp.maximum(deg, 1e-12)), 0.0)
    ds_ref[...] = dinv * S

    @pl.when(i == 0)
    def _init():
        out_ref[...] = jnp.zeros_like(out_ref)
        ss_ref[0, 0] = 0.0

    out_ref[...] += lax.dot_general(
        S, mat_y, (((0,), (0,)), ((), ())), preferred_element_type=F32)
    ss_ref[0, 0] += jnp.sum(S * S)


def _run_k2(agg0, agg1, xp, w1p_root, b1p, w1e_root, b1e):
    blk = 1000
    return pl.pallas_call(
        _k2_body,
        grid=(N // blk,),
        in_specs=[
            pl.BlockSpec((blk, W), lambda i: (i, 0)),
            pl.BlockSpec((blk, W), lambda i: (i, 0)),
            pl.BlockSpec((blk, 5), lambda i: (i, 0)),
            pl.BlockSpec((5, 128), lambda i: (0, 0)),
            pl.BlockSpec((1, 128), lambda i: (0, 0)),
            pl.BlockSpec((5, 64), lambda i: (0, 0)),
            pl.BlockSpec((1, 64), lambda i: (0, 0)),
        ],
        out_specs=[
            pl.BlockSpec((blk, 128), lambda i: (i, 0)),
            pl.BlockSpec((128, 64), lambda i: (0, 0)),
            pl.BlockSpec(memory_space=pltpu.SMEM),
        ],
        out_shape=[
            jax.ShapeDtypeStruct((NP, 128), F32),
            jax.ShapeDtypeStruct((128, 64), F32),
            jax.ShapeDtypeStruct((1, 1), F32),
        ],
    )(agg0, agg1, xp, w1p_root, b1p, w1e_root, b1e)


def _scb_body(ds, sd3, wmat, za_o, zb_o,
              zacc, rows_a, rows_b, sdv_a, sdv_b, wvm_a, wvm_b,
              sem_a, sem_b):
    c = lax.axis_index("c")
    s = lax.axis_index("s")
    zf = jnp.zeros((16,), F32)

    def _zero_row(r, _):
        for j in range(8):
            rows_a[r, pl.ds(j * 16, 16)] = zf
        return 0
    lax.fori_loop(0, CHB, _zero_row, 0)

    def _zero_acc(j, _):
        pltpu.sync_copy(rows_a, zacc.at[pl.ds(s * STRIPE + j * CHB, CHB)])
        return 0
    lax.fori_loop(0, STRIPE // CHB, _zero_acc, 0)
    plsc.subcore_barrier()

    wid = s * 2 + c

    def _prep_fire(sdv, wvm, rows, sem, gidx):
        pltpu.sync_copy(sd3.at[gidx], sdv)
        pltpu.sync_copy(wmat.at[pl.ds(gidx * CHB, CHB)], wvm)
        pltpu.async_copy(ds.at[sdv.at[1]], rows, sem)

    def _finish(sdv, wvm, rows, sem):
        pltpu.make_async_copy(ds.at[sdv.at[1]], rows, sem).wait()

        def _scale(t, _):
            for u in range(4):
                e = t * 4 + u
                spl = wvm[e, :]
                for k in range(8):
                    v = rows[e, pl.ds(k * 16, 16)]
                    rows[e, pl.ds(k * 16, 16)] = v * spl
            return 0
        lax.fori_loop(0, CHB // 4, _scale, 0)

        pltpu.sync_copy(rows, zacc.at[sdv.at[0]], add=True)

    g00 = wid * CPWB
    _prep_fire(sdv_a, wvm_a, rows_a, sem_a, g00)

    def _blk(h, _):
        g0 = g00 + 2 * h
        _prep_fire(sdv_b, wvm_b, rows_b, sem_b, g0 + 1)
        _finish(sdv_a, wvm_a, rows_a, sem_a)

        @pl.when(h + 1 < CPWB // 2)
        def _next():
            _prep_fire(sdv_a, wvm_a, rows_a, sem_a, g0 + 2)

        _finish(sdv_b, wvm_b, rows_b, sem_b)
        return 0
    lax.fori_loop(0, CPWB // 2, _blk, 0)
    plsc.subcore_barrier()

    def _flush(j, _):
        r0 = s * STRIPE + j * CH

        @pl.when(c == 0)
        def _f0():
            pltpu.sync_copy(zacc.at[pl.ds(r0, CH)], za_o.at[pl.ds(r0, CH)])

        @pl.when(c == 1)
        def _f1():
            pltpu.sync_copy(zacc.at[pl.ds(r0, CH)], zb_o.at[pl.ds(r0, CH)])
        return 0
    lax.fori_loop(0, STRIPE // CH, _flush, 0)


def _run_scb(ds, sd3, wmat):
    mesh = plsc.VectorSubcoreMesh(core_axis_name="c", subcore_axis_name="s")
    f = pl.kernel(
        _scb_body, mesh=mesh,
        out_type=[
            jax.ShapeDtypeStruct((NP, 128), F32),
            jax.ShapeDtypeStruct((NP, 128), F32),
        ],
        scratch_types=[
            pltpu.VMEM_SHARED((NP, 128), F32),
            pltpu.VMEM((CHB, 128), F32),
            pltpu.VMEM((CHB, 128), F32),
            pltpu.VMEM((3, CHB), I32),
            pltpu.VMEM((3, CHB), I32),
            pltpu.VMEM((CHB, 16), F32),
            pltpu.VMEM((CHB, 16), F32),
            pltpu.SemaphoreType.DMA,
            pltpu.SemaphoreType.DMA,
        ],
    )
    return f(ds, sd3, wmat)


def _k3_body(ds_ref, za_ref, zb_ref, out_ref, ss_ref,
             w2ew_ref, w2er_ref, b2e_ref, l1w_ref, l1b_ref, l2w_ref,
             l2b_ref, logp_ref, reg_ref):
    i = pl.program_id(0)
    ng = pl.num_programs(0)

    @pl.when(i == 0)
    def _init():
        reg_ref[0, 0] = 0.0

    reg_ref[0, 0] += jnp.sum(ds_ref[...] * (za_ref[...] + zb_ref[...]))

    @pl.when(i == ng - 1)
    def _final():
        reg_ref[0, 0] = (ss_ref[0, 0] - reg_ref[0, 0]) / float(N)
        o = out_ref[...]
        t = jnp.dot(o, w2ew_ref[...], preferred_element_type=F32)
        me = jnp.sum(t, axis=0, keepdims=True) / 128.0
        my2 = jnp.maximum(
            me + jnp.dot(o, w2er_ref[...], preferred_element_type=F32)
            + b2e_ref[...], 0.0)
        out2 = jnp.sum(my2, axis=0, keepdims=True)
        h = jnp.maximum(
            jnp.dot(out2, l1w_ref[...], preferred_element_type=F32)
            + l1b_ref[...], 0.0)
        lg = jnp.dot(h, l2w_ref[...], preferred_element_type=F32) \
            + l2b_ref[...]
        m = jnp.max(lg)
        logp_ref[...] = lg - (m + jnp.log(jnp.sum(jnp.exp(lg - m))))


def _run_k3(ds, za, zb, out128, ss, w2e_w, w2e_root, b2e,
            lin1_w, lin1_b, lin2_w, lin2_b):
    blk = 1000
    return pl.pallas_call(
        _k3_body,
        grid=(N // blk,),
        in_specs=[
            pl.BlockSpec((blk, 128), lambda i: (i, 0)),
            pl.BlockSpec((blk, 128), lambda i: (i, 0)),
            pl.BlockSpec((blk, 128), lambda i: (i, 0)),
            pl.BlockSpec((128, 64), lambda i: (0, 0)),
            pl.BlockSpec(memory_space=pltpu.SMEM),
            pl.BlockSpec((64, 64), lambda i: (0, 0)),
            pl.BlockSpec((64, 64), lambda i: (0, 0)),
            pl.BlockSpec((1, 64), lambda i: (0, 0)),
            pl.BlockSpec((64, 256), lambda i: (0, 0)),
            pl.BlockSpec((1, 256), lambda i: (0, 0)),
            pl.BlockSpec((256, 8), lambda i: (0, 0)),
            pl.BlockSpec((1, 8), lambda i: (0, 0)),
        ],
        out_specs=[
            pl.BlockSpec((1, 8), lambda i: (0, 0)),
            pl.BlockSpec(memory_space=pltpu.SMEM),
        ],
        out_shape=[
            jax.ShapeDtypeStruct((1, 8), F32),
            jax.ShapeDtypeStruct((1, 1), F32),
        ],
    )(ds, za, zb, out128, ss, w2e_w, w2e_root, b2e,
      lin1_w, lin1_b, lin2_w, lin2_b)


def kernel(x, edge_index, edge_wht,
           w1p_w, w1p_root, w1p_b, w1e_w, w1e_root, w1e_b,
           w2p_w, w2p_root, w2p_b, w2e_w, w2e_root, w2e_b,
           lin1_w, lin1_b, lin2_w, lin2_b):
    src = edge_index[0]
    dst = edge_index[1]
    w = edge_wht.reshape(-1)
    pad = EP - E
    spread = (N + (jnp.arange(pad, dtype=I32) % (NP - N))).astype(I32)
    srcp = jnp.concatenate([src.astype(I32), spread])
    dstp = jnp.concatenate([dst.astype(I32), spread])
    wp = jnp.concatenate([w, jnp.zeros((pad,), F32)])
    wmat = jnp.broadcast_to(wp[:, None], (EP, 16)) * jnp.ones((EP, 16), F32)
    srcb = srcp.reshape(-1, CHB)
    dstb = dstp.reshape(-1, CHB)
    sd3 = jnp.stack([srcb, dstb, srcb], axis=1)
    xp = jnp.pad(x, ((0, NP - N), (0, 0)))

    (tt,) = _build_tables(xp, w1p_w, w1e_w)
    agg0, agg1 = _run_sca(tt, srcp, dstp, wmat)
    ds, out128, ss = _run_k2(
        agg0, agg1, xp, w1p_root, w1p_b.reshape(1, 128),
        w1e_root, w1e_b.reshape(1, 64))
    za, zb = _run_scb(ds, sd3, wmat)
    logp, reg = _run_k3(
        ds, za, zb, out128, ss, w2e_w, w2e_root,
        w2e_b.reshape(1, 64), lin1_w, lin1_b.reshape(1, 256),
        lin2_w, lin2_b.reshape(1, 8))
    return (logp, reg.reshape(()))


# speedup vs baseline: 4.4446x; 1.6100x over previous
import jax
import jax.numpy as jnp
from jax import lax
from jax.experimental import pallas as pl
from jax.experimental.pallas import tpu as pltpu
from jax.experimental.pallas import tpu_sc as plsc

N = 10000
NP = 10240
E = 320000
CH = 64
NSUB = 16
CPT = 316
CHB = 64
CPWB = 158
EP = NSUB * CPT * CH
W = 128
STRIPE = NP // NSUB
F32 = jnp.float32
I32 = jnp.int32


def _k1_body(x_ref, wp_ref, we_ref, tt_ref):
    i = pl.program_id(0)
    xb = x_ref[...]
    tp = jnp.dot(xb, wp_ref[...], preferred_element_type=F32)
    te = jnp.dot(xb, we_ref[...], preferred_element_type=F32)
    nrows = xb.shape[0]
    t0 = jnp.concatenate([tp[:, :112], jnp.zeros((nrows, 16), F32)], axis=1)
    t1 = jnp.concatenate(
        [tp[:, 112:], te, jnp.ones((nrows, 1), F32),
         jnp.zeros((nrows, 47), F32)], axis=1)
    tt_ref[...] = jnp.where(i < NP // 1280, t0, t1)


def _build_tables(xp, w1p_w, w1e_w):
    blk = 1280
    nb = NP // blk
    return pl.pallas_call(
        _k1_body,
        grid=(2 * nb,),
        in_specs=[
            pl.BlockSpec((blk, 5), lambda i: (lax.rem(i, nb), 0)),
            pl.BlockSpec((5, 128), lambda i: (0, 0)),
            pl.BlockSpec((5, 64), lambda i: (0, 0)),
        ],
        out_specs=[
            pl.BlockSpec((blk, W), lambda i: (i, 0)),
        ],
        out_shape=[
            jax.ShapeDtypeStruct((2 * NP, W), F32),
        ],
    )(xp, w1p_w, w1e_w)


DCOL = 112


def _sca_body(tt, sd3, wmat, agg0_o, agg1_o,
              acc, rows_a, rows_b, sdv_a, sdv_b, wrow, wfull,
              sem_a, sem_b):
    c = lax.axis_index("c")
    s = lax.axis_index("s")
    zf = jnp.zeros((16,), F32)
    coff = c * NP

    def _zero_row(r, _):
        for j in range(W // 16):
            rows_a[r, pl.ds(j * 16, 16)] = zf
            wfull[r, pl.ds(j * 16, 16)] = zf
        return 0
    lax.fori_loop(0, CH, _zero_row, 0)

    def _zero_acc(j, _):
        r0 = s * STRIPE + j * CH
        pltpu.sync_copy(rows_a, acc.at[pl.ds(r0, CH)])
        return 0
    lax.fori_loop(0, STRIPE // CH, _zero_acc, 0)
    plsc.subcore_barrier()

    def _prep_fire(sdv, rows, sem, g):
        gidx = s * CPT + g
        pltpu.sync_copy(sd3.at[gidx], sdv)

        deg_mine = jnp.logical_xor(g >= CPT // 2, c == 0)

        @pl.when(deg_mine)
        def _deg():
            pltpu.sync_copy(wmat.at[pl.ds(gidx * CH, CH)], wrow)

            def _exp(r, _):
                wfull[r, pl.ds(DCOL, 16)] = wrow[r, :]
                return 0
            lax.fori_loop(0, CH, _exp, 0)
            pltpu.sync_copy(wfull, acc.at[sdv.at[2]], add=True)

        for j in range(CH // 16):
            v = sdv[0, pl.ds(j * 16, 16)]
            sdv[0, pl.ds(j * 16, 16)] = v + coff
        pltpu.async_copy(tt.at[sdv.at[0]], rows, sem)

    def _finish(sdv, rows, sem):
        pltpu.make_async_copy(tt.at[sdv.at[0]], rows, sem).wait()
        pltpu.sync_copy(rows, acc.at[sdv.at[1]], add=True)

    _prep_fire(sdv_a, rows_a, sem_a, 0)

    def _blk(h, _):
        _prep_fire(sdv_b, rows_b, sem_b, 2 * h + 1)
        _finish(sdv_a, rows_a, sem_a)

        @pl.when(h + 1 < CPT // 2)
        def _next():
            _prep_fire(sdv_a, rows_a, sem_a, 2 * h + 2)

        _finish(sdv_b, rows_b, sem_b)
        return 0
    lax.fori_loop(0, CPT // 2, _blk, 0)
    plsc.subcore_barrier()

    def _flush(j, _):
        r0 = s * STRIPE + j * CH

        @pl.when(c == 0)
        def _f0():
            pltpu.sync_copy(acc.at[pl.ds(r0, CH)], agg0_o.at[pl.ds(r0, CH)])

        @pl.when(c == 1)
        def _f1():
            pltpu.sync_copy(acc.at[pl.ds(r0, CH)], agg1_o.at[pl.ds(r0, CH)])
        return 0
    lax.fori_loop(0, STRIPE // CH, _flush, 0)


def _run_sca(tt, sd3, wmat):
    mesh = plsc.VectorSubcoreMesh(core_axis_name="c", subcore_axis_name="s")
    f = pl.kernel(
        _sca_body, mesh=mesh,
        out_type=[
            jax.ShapeDtypeStruct((NP, W), F32),
            jax.ShapeDtypeStruct((NP, W), F32),
        ],
        scratch_types=[
            pltpu.VMEM_SHARED((NP, W), F32),
            pltpu.VMEM((CH, W), F32),
            pltpu.VMEM((CH, W), F32),
            pltpu.VMEM((3, CH), I32),
            pltpu.VMEM((3, CH), I32),
            pltpu.VMEM((CH, 16), F32),
            pltpu.VMEM((CH, W), F32),
            pltpu.SemaphoreType.DMA,
            pltpu.SemaphoreType.DMA,
        ],
    )
    return f(tt, sd3, wmat)


def _k2_body(a0_ref, a1_ref, x_ref, wpr_ref, bp_ref, wer_ref,
             be_ref, ds_ref, out_ref, ss_ref):
    i = pl.program_id(0)
    a0 = a0_ref[...]
    a1 = a1_ref[...]
    x = x_ref[...]
    cnt = a1[:, 80:81]
    deg = a0[:, DCOL:DCOL + 1] + a1[:, DCOL:DCOL + 1]
    cntc = jnp.maximum(cnt, 1.0)
    aggP = jnp.concatenate([a0[:, :112], a1[:, :16]], axis=1)
    mat_s = aggP / cntc + jnp.dot(x, wpr_ref[...], preferred_element_type=F32) \
        + bp_ref[...]
    m = jnp.max(mat_s, axis=1, keepdims=True)
    e = jnp.exp(mat_s - m)
    S = e / jnp.sum(e, axis=1, keepdims=True)
    mat_y = jnp.maximum(
        a1[:, 16:80] / cntc
        + jnp.dot(x, wer_ref[...], preferred_element_type=F32)
        + be_ref[...], 0.0)
    dinv = jnp.where(deg > 0.0, 1.0 / jnp.sqrt(jnp.maximum(deg, 1e-12)), 0.0)
    ds_ref[...] = dinv * S

    @pl.when(i == 0)
    def _init():
        out_ref[...] = jnp.zeros_like(out_ref)
        ss_ref[0, 0] = 0.0

    out_ref[...] += lax.dot_general(
        S, mat_y, (((0,), (0,)), ((), ())), preferred_element_type=F32)
    ss_ref[0, 0] += jnp.sum(S * S)


def _run_k2(agg0, agg1, xp, w1p_root, b1p, w1e_root, b1e):
    blk = 1000
    return pl.pallas_call(
        _k2_body,
        grid=(N // blk,),
        in_specs=[
            pl.BlockSpec((blk, W), lambda i: (i, 0)),
            pl.BlockSpec((blk, W), lambda i: (i, 0)),
            pl.BlockSpec((blk, 5), lambda i: (i, 0)),
            pl.BlockSpec((5, 128), lambda i: (0, 0)),
            pl.BlockSpec((1, 128), lambda i: (0, 0)),
            pl.BlockSpec((5, 64), lambda i: (0, 0)),
            pl.BlockSpec((1, 64), lambda i: (0, 0)),
        ],
        out_specs=[
            pl.BlockSpec((blk, 128), lambda i: (i, 0)),
            pl.BlockSpec((128, 64), lambda i: (0, 0)),
            pl.BlockSpec(memory_space=pltpu.SMEM),
        ],
        out_shape=[
            jax.ShapeDtypeStruct((NP, 128), F32),
            jax.ShapeDtypeStruct((128, 64), F32),
            jax.ShapeDtypeStruct((1, 1), F32),
        ],
    )(agg0, agg1, xp, w1p_root, b1p, w1e_root, b1e)


def _scb_body(ds, sd3, wmat, za_o, zb_o,
              zacc, rows_a, rows_b, sdv_a, sdv_b, wvm_a, wvm_b,
              sem_a, sem_b):
    c = lax.axis_index("c")
    s = lax.axis_index("s")
    zf = jnp.zeros((16,), F32)

    def _zero_row(r, _):
        for j in range(8):
            rows_a[r, pl.ds(j * 16, 16)] = zf
        return 0
    lax.fori_loop(0, CHB, _zero_row, 0)

    def _zero_acc(j, _):
        pltpu.sync_copy(rows_a, zacc.at[pl.ds(s * STRIPE + j * CHB, CHB)])
        return 0
    lax.fori_loop(0, STRIPE // CHB, _zero_acc, 0)
    plsc.subcore_barrier()

    wid = s * 2 + c

    def _prep_fire(sdv, wvm, rows, sem, gidx):
        pltpu.sync_copy(sd3.at[gidx], sdv)
        pltpu.sync_copy(wmat.at[pl.ds(gidx * CHB, CHB)], wvm)
        pltpu.async_copy(ds.at[sdv.at[1]], rows, sem)

    def _finish(sdv, wvm, rows, sem):
        pltpu.make_async_copy(ds.at[sdv.at[1]], rows, sem).wait()

        def _scale(t, _):
            for u in range(4):
                e = t * 4 + u
                spl = wvm[e, :]
                for k in range(8):
                    v = rows[e, pl.ds(k * 16, 16)]
                    rows[e, pl.ds(k * 16, 16)] = v * spl
            return 0
        lax.fori_loop(0, CHB // 4, _scale, 0)

        pltpu.sync_copy(rows, zacc.at[sdv.at[0]], add=True)

    g00 = wid * CPWB
    _prep_fire(sdv_a, wvm_a, rows_a, sem_a, g00)

    def _blk(h, _):
        g0 = g00 + 2 * h
        _prep_fire(sdv_b, wvm_b, rows_b, sem_b, g0 + 1)
        _finish(sdv_a, wvm_a, rows_a, sem_a)

        @pl.when(h + 1 < CPWB // 2)
        def _next():
            _prep_fire(sdv_a, wvm_a, rows_a, sem_a, g0 + 2)

        _finish(sdv_b, wvm_b, rows_b, sem_b)
        return 0
    lax.fori_loop(0, CPWB // 2, _blk, 0)
    plsc.subcore_barrier()

    def _flush(j, _):
        r0 = s * STRIPE + j * CH

        @pl.when(c == 0)
        def _f0():
            pltpu.sync_copy(zacc.at[pl.ds(r0, CH)], za_o.at[pl.ds(r0, CH)])

        @pl.when(c == 1)
        def _f1():
            pltpu.sync_copy(zacc.at[pl.ds(r0, CH)], zb_o.at[pl.ds(r0, CH)])
        return 0
    lax.fori_loop(0, STRIPE // CH, _flush, 0)


def _run_scb(ds, sd3, wmat):
    mesh = plsc.VectorSubcoreMesh(core_axis_name="c", subcore_axis_name="s")
    f = pl.kernel(
        _scb_body, mesh=mesh,
        out_type=[
            jax.ShapeDtypeStruct((NP, 128), F32),
            jax.ShapeDtypeStruct((NP, 128), F32),
        ],
        scratch_types=[
            pltpu.VMEM_SHARED((NP, 128), F32),
            pltpu.VMEM((CHB, 128), F32),
            pltpu.VMEM((CHB, 128), F32),
            pltpu.VMEM((3, CHB), I32),
            pltpu.VMEM((3, CHB), I32),
            pltpu.VMEM((CHB, 16), F32),
            pltpu.VMEM((CHB, 16), F32),
            pltpu.SemaphoreType.DMA,
            pltpu.SemaphoreType.DMA,
        ],
    )
    return f(ds, sd3, wmat)


def _k3_body(ds_ref, za_ref, zb_ref, out_ref, ss_ref,
             w2ew_ref, w2er_ref, b2e_ref, l1w_ref, l1b_ref, l2w_ref,
             l2b_ref, logp_ref, reg_ref):
    i = pl.program_id(0)
    ng = pl.num_programs(0)

    @pl.when(i == 0)
    def _init():
        reg_ref[0, 0] = 0.0

    reg_ref[0, 0] += jnp.sum(ds_ref[...] * (za_ref[...] + zb_ref[...]))

    @pl.when(i == ng - 1)
    def _final():
        reg_ref[0, 0] = (ss_ref[0, 0] - reg_ref[0, 0]) / float(N)
        o = out_ref[...]
        t = jnp.dot(o, w2ew_ref[...], preferred_element_type=F32)
        me = jnp.sum(t, axis=0, keepdims=True) / 128.0
        my2 = jnp.maximum(
            me + jnp.dot(o, w2er_ref[...], preferred_element_type=F32)
            + b2e_ref[...], 0.0)
        out2 = jnp.sum(my2, axis=0, keepdims=True)
        h = jnp.maximum(
            jnp.dot(out2, l1w_ref[...], preferred_element_type=F32)
            + l1b_ref[...], 0.0)
        lg = jnp.dot(h, l2w_ref[...], preferred_element_type=F32) \
            + l2b_ref[...]
        m = jnp.max(lg)
        logp_ref[...] = lg - (m + jnp.log(jnp.sum(jnp.exp(lg - m))))


def _run_k3(ds, za, zb, out128, ss, w2e_w, w2e_root, b2e,
            lin1_w, lin1_b, lin2_w, lin2_b):
    blk = 1000
    return pl.pallas_call(
        _k3_body,
        grid=(N // blk,),
        in_specs=[
            pl.BlockSpec((blk, 128), lambda i: (i, 0)),
            pl.BlockSpec((blk, 128), lambda i: (i, 0)),
            pl.BlockSpec((blk, 128), lambda i: (i, 0)),
            pl.BlockSpec((128, 64), lambda i: (0, 0)),
            pl.BlockSpec(memory_space=pltpu.SMEM),
            pl.BlockSpec((64, 64), lambda i: (0, 0)),
            pl.BlockSpec((64, 64), lambda i: (0, 0)),
            pl.BlockSpec((1, 64), lambda i: (0, 0)),
            pl.BlockSpec((64, 256), lambda i: (0, 0)),
            pl.BlockSpec((1, 256), lambda i: (0, 0)),
            pl.BlockSpec((256, 8), lambda i: (0, 0)),
            pl.BlockSpec((1, 8), lambda i: (0, 0)),
        ],
        out_specs=[
            pl.BlockSpec((1, 8), lambda i: (0, 0)),
            pl.BlockSpec(memory_space=pltpu.SMEM),
        ],
        out_shape=[
            jax.ShapeDtypeStruct((1, 8), F32),
            jax.ShapeDtypeStruct((1, 1), F32),
        ],
    )(ds, za, zb, out128, ss, w2e_w, w2e_root, b2e,
      lin1_w, lin1_b, lin2_w, lin2_b)


def kernel(x, edge_index, edge_wht,
           w1p_w, w1p_root, w1p_b, w1e_w, w1e_root, w1e_b,
           w2p_w, w2p_root, w2p_b, w2e_w, w2e_root, w2e_b,
           lin1_w, lin1_b, lin2_w, lin2_b):
    src = edge_index[0]
    dst = edge_index[1]
    w = edge_wht.reshape(-1)
    pad = EP - E
    spread = (N + (jnp.arange(pad, dtype=I32) % (NP - N))).astype(I32)
    srcp = jnp.concatenate([src.astype(I32), spread])
    dstp = jnp.concatenate([dst.astype(I32), spread])
    wp = jnp.concatenate([w, jnp.zeros((pad,), F32)])
    wmat = jnp.broadcast_to(wp[:, None], (EP, 16)) * jnp.ones((EP, 16), F32)
    srcb = srcp.reshape(-1, CHB)
    dstb = dstp.reshape(-1, CHB)
    sd3 = jnp.stack([srcb, dstb, srcb], axis=1)
    xp = jnp.pad(x, ((0, NP - N), (0, 0)))

    (tt,) = _build_tables(xp, w1p_w, w1e_w)
    agg0, agg1 = _run_sca(tt, sd3, wmat)
    ds, out128, ss = _run_k2(
        agg0, agg1, xp, w1p_root, w1p_b.reshape(1, 128),
        w1e_root, w1e_b.reshape(1, 64))
    za, zb = _run_scb(ds, sd3, wmat)
    logp, reg = _run_k3(
        ds, za, zb, out128, ss, w2e_w, w2e_root,
        w2e_b.reshape(1, 64), lin1_w, lin1_b.reshape(1, 256),
        lin2_w, lin2_b.reshape(1, 8))
    return (logp, reg.reshape(()))
